# Initial kernel scaffold; baseline (speedup 1.0000x reference)
import jax
import jax.numpy as jnp
from jax.experimental import pallas as pl


def kernel(hidden_states, classifier_weight, e_score_correction_bias):
    raise NotImplementedError("write your pallas kernel here")


# speedup vs baseline: 1.5211x; 1.5211x over previous
import functools

import jax
import jax.numpy as jnp
from jax.experimental import pallas as pl
from jax.experimental.pallas import tpu as pltpu

HIDDEN = 2048
NUM_EXPERTS = 64
TOP_K = 8
ROUTED_SCALING_FACTOR = 1.5
TOKENS = 8192
BLK = 1024


def _router_body(hs_ref, wt_ref, bias_ref, idx_ref, w_ref):
    hs = hs_ref[...]
    logits = jnp.dot(hs, wt_ref[...], preferred_element_type=jnp.float32)
    m = jnp.max(logits, axis=-1, keepdims=True)
    e = jnp.exp(logits - m)
    scores = e / jnp.sum(e, axis=-1, keepdims=True)
    biased = scores + bias_ref[...]

    col = jax.lax.broadcasted_iota(jnp.int32, (BLK, NUM_EXPERTS), 1)
    work = biased
    idx_cols = []
    w_cols = []
    for _ in range(TOP_K):
        mx = jnp.max(work, axis=-1, keepdims=True)
        eq = work == mx
        pick = jnp.min(jnp.where(eq, col, NUM_EXPERTS), axis=-1, keepdims=True)
        sel = col == pick
        wj = jnp.sum(jnp.where(sel, scores, 0.0), axis=-1, keepdims=True)
        idx_cols.append(pick)
        w_cols.append(wj * ROUTED_SCALING_FACTOR)
        work = jnp.where(sel, -jnp.inf, work)
    idx_ref[...] = jnp.concatenate(idx_cols, axis=1)
    w_ref[...] = jnp.concatenate(w_cols, axis=1)


@jax.jit
def _router(hs, wt, bias):
    grid = TOKENS // BLK
    return pl.pallas_call(
        _router_body,
        grid=(grid,),
        in_specs=[
            pl.BlockSpec((BLK, HIDDEN), lambda i: (i, 0)),
            pl.BlockSpec((HIDDEN, NUM_EXPERTS), lambda i: (0, 0)),
            pl.BlockSpec((1, NUM_EXPERTS), lambda i: (0, 0)),
        ],
        out_specs=[
            pl.BlockSpec((BLK, TOP_K), lambda i: (i, 0)),
            pl.BlockSpec((BLK, TOP_K), lambda i: (i, 0)),
        ],
        out_shape=[
            jax.ShapeDtypeStruct((TOKENS, TOP_K), jnp.int32),
            jax.ShapeDtypeStruct((TOKENS, TOP_K), jnp.float32),
        ],
    )(hs, wt, bias)


def kernel(hidden_states, classifier_weight, e_score_correction_bias):
    hs = hidden_states.reshape(-1, HIDDEN).astype(jnp.float32)
    wt = classifier_weight.T
    bias = e_score_correction_bias.reshape(1, NUM_EXPERTS)
    idx, w = _router(hs, wt, bias)
    return idx, w


# speedup vs baseline: 1.7092x; 1.1237x over previous
import functools

import jax
import jax.numpy as jnp
from jax import lax
from jax.experimental import pallas as pl
from jax.experimental.pallas import tpu as pltpu
from jax.experimental.pallas import tpu_sc as plsc

HIDDEN = 2048
NUM_EXPERTS = 64
TOP_K = 8
ROUTED_SCALING_FACTOR = 1.5
TOKENS = 8192
BLK = 1024

NC, NS, L = 2, 16, 16
NW = NC * NS
TPW = TOKENS // NW
NG = TPW // L


def _tc_body(hs_ref, w_ref, bias_ref, out_ref):
    lt = lax.dot_general(
        w_ref[...], hs_ref[...],
        dimension_numbers=(((1,), (1,)), ((), ())),
        preferred_element_type=jnp.float32,
    )
    m = jnp.max(lt, axis=0, keepdims=True)
    e = jnp.exp(lt - m)
    scores = e / jnp.sum(e, axis=0, keepdims=True)
    biased = scores + bias_ref[...]
    for j in range(BLK // TPW):
        out_ref[j] = biased[:, j * TPW:(j + 1) * TPW]


@jax.jit
def _tc_scores(hs, w, bias):
    grid = TOKENS // BLK
    return pl.pallas_call(
        _tc_body,
        grid=(grid,),
        in_specs=[
            pl.BlockSpec((BLK, HIDDEN), lambda i: (i, 0)),
            pl.BlockSpec((NUM_EXPERTS, HIDDEN), lambda i: (0, 0)),
            pl.BlockSpec((NUM_EXPERTS, 1), lambda i: (0, 0)),
        ],
        out_specs=pl.BlockSpec((BLK // TPW, NUM_EXPERTS, TPW),
                               lambda i: (i, 0, 0)),
        out_shape=jax.ShapeDtypeStruct((NW, NUM_EXPERTS, TPW), jnp.float32),
    )(hs, w, bias)


def _sc_body(bt_hbm, bias_hbm, idx_hbm, w_hbm, bt_v, bias_v, idx_v, w_v):
    wid = lax.axis_index("s") * NC + lax.axis_index("c")
    pltpu.sync_copy(bt_hbm.at[wid], bt_v)
    pltpu.sync_copy(bias_hbm, bias_v)
    neg_inf = jnp.full((L,), -jnp.inf, jnp.float32)
    zero_i = jnp.zeros((L,), jnp.int32)
    bias_regs = [bias_v[pl.ds(k * L, L)] for k in range(NUM_EXPERTS // L)]

    def group_body(g, _):
        col0 = g * L

        def expert_body(e, carry):
            rs = list(carry[:TOP_K])
            ixs = list(carry[TOP_K:])
            v = bt_v[e, pl.ds(col0, L)]
            iv = jnp.broadcast_to(e.astype(jnp.int32), (L,))
            for j in range(TOP_K):
                p = v > rs[j]
                rs[j], v = jnp.where(p, v, rs[j]), jnp.where(p, rs[j], v)
                ixs[j], iv = jnp.where(p, iv, ixs[j]), jnp.where(p, ixs[j], iv)
            return tuple(rs) + tuple(ixs)

        carry = tuple([neg_inf] * TOP_K) + tuple([zero_i] * TOP_K)
        carry = lax.fori_loop(0, NUM_EXPERTS, expert_body, carry)
        rs = carry[:TOP_K]
        ixs = carry[TOP_K:]
        for j in range(TOP_K):
            ix = ixs[j]
            lo = ix & (L - 1)
            hi = ix >> 4
            b = bias_regs[0].at[lo].get(mode="promise_in_bounds")
            for k in range(1, NUM_EXPERTS // L):
                gk = bias_regs[k].at[lo].get(mode="promise_in_bounds")
                b = jnp.where(hi == k, gk, b)
            wj = (rs[j] - b) * ROUTED_SCALING_FACTOR
            idx_v[j, pl.ds(col0, L)] = ix
            w_v[j, pl.ds(col0, L)] = wj
        return 0

    lax.fori_loop(0, NG, group_body, 0)
    pltpu.sync_copy(idx_v, idx_hbm.at[:, pl.ds(wid * TPW, TPW)])
    pltpu.sync_copy(w_v, w_hbm.at[:, pl.ds(wid * TPW, TPW)])


@jax.jit
def _sc_topk(bt, bias):
    mesh = plsc.VectorSubcoreMesh(core_axis_name="c", subcore_axis_name="s")
    return pl.kernel(
        _sc_body,
        out_type=[
            jax.ShapeDtypeStruct((TOP_K, TOKENS), jnp.int32),
            jax.ShapeDtypeStruct((TOP_K, TOKENS), jnp.float32),
        ],
        mesh=mesh,
        scratch_types=[
            pltpu.VMEM((NUM_EXPERTS, TPW), jnp.float32),
            pltpu.VMEM((NUM_EXPERTS,), jnp.float32),
            pltpu.VMEM((TOP_K, TPW), jnp.int32),
            pltpu.VMEM((TOP_K, TPW), jnp.float32),
        ],
    )(bt, bias)


def kernel(hidden_states, classifier_weight, e_score_correction_bias):
    hs = hidden_states.reshape(-1, HIDDEN).astype(jnp.float32)
    bias_col = e_score_correction_bias.reshape(NUM_EXPERTS, 1)
    bt = _tc_scores(hs, classifier_weight, bias_col)
    idx_rm, w_rm = _sc_topk(bt, e_score_correction_bias)
    return idx_rm.T, w_rm.T


# speedup vs baseline: 1.7360x; 1.0157x over previous
import functools

import jax
import jax.numpy as jnp
from jax import lax
from jax.experimental import pallas as pl
from jax.experimental.pallas import tpu as pltpu
from jax.experimental.pallas import tpu_sc as plsc

HIDDEN = 2048
NUM_EXPERTS = 64
TOP_K = 8
ROUTED_SCALING_FACTOR = 1.5
TOKENS = 8192
BLK = 1024

NC, NS, L = 2, 16, 16
NW = NC * NS
TPW = TOKENS // NW
NG = TPW // L


def _tc_body(hs_ref, w_ref, bias_ref, out_ref):
    lt = lax.dot_general(
        w_ref[...], hs_ref[...],
        dimension_numbers=(((1,), (1,)), ((), ())),
        preferred_element_type=jnp.float32,
    )
    m = jnp.max(lt, axis=0, keepdims=True)
    e = jnp.exp(lt - m)
    scores = e / jnp.sum(e, axis=0, keepdims=True)
    biased = scores + bias_ref[...]
    for j in range(BLK // TPW):
        out_ref[j] = biased[:, j * TPW:(j + 1) * TPW]


@jax.jit
def _tc_scores(hs, w, bias):
    grid = TOKENS // BLK
    return pl.pallas_call(
        _tc_body,
        grid=(grid,),
        in_specs=[
            pl.BlockSpec((BLK, HIDDEN), lambda i: (i, 0)),
            pl.BlockSpec((NUM_EXPERTS, HIDDEN), lambda i: (0, 0)),
            pl.BlockSpec((NUM_EXPERTS, 1), lambda i: (0, 0)),
        ],
        out_specs=pl.BlockSpec((BLK // TPW, NUM_EXPERTS, TPW),
                               lambda i: (i, 0, 0)),
        out_shape=jax.ShapeDtypeStruct((NW, NUM_EXPERTS, TPW), jnp.float32),
    )(hs, w, bias)


def _sc_body(bt_hbm, bias_hbm, idx_hbm, w_hbm, bt_v, bias_v, idx_v, w_v):
    wid = lax.axis_index("s") * NC + lax.axis_index("c")
    pltpu.sync_copy(bt_hbm.at[wid], bt_v)
    pltpu.sync_copy(bias_hbm, bias_v)
    neg_inf = jnp.full((L,), -jnp.inf, jnp.float32)
    zero_i = jnp.zeros((L,), jnp.int32)
    bias_regs = [bias_v[pl.ds(k * L, L)] for k in range(NUM_EXPERTS // L)]

    def group_body(g, _):
        col0 = g * (2 * L)

        def expert_body(e, carry):
            rs = [list(carry[0][q]) for q in range(2)]
            ixs = [list(carry[1][q]) for q in range(2)]
            vs = [bt_v[e, pl.ds(col0 + q * L, L)] for q in range(2)]
            iv0 = jnp.broadcast_to(e.astype(jnp.int32), (L,))
            ivs = [iv0, iv0]
            for j in range(TOP_K):
                for q in range(2):
                    p = vs[q] > rs[q][j]
                    rs[q][j], vs[q] = (jnp.where(p, vs[q], rs[q][j]),
                                       jnp.where(p, rs[q][j], vs[q]))
                    ixs[q][j], ivs[q] = (jnp.where(p, ivs[q], ixs[q][j]),
                                         jnp.where(p, ixs[q][j], ivs[q]))
            return (tuple(tuple(r) for r in rs), tuple(tuple(i) for i in ixs))

        carry = (tuple(tuple([neg_inf] * TOP_K) for _ in range(2)),
                 tuple(tuple([zero_i] * TOP_K) for _ in range(2)))
        carry = lax.fori_loop(0, NUM_EXPERTS, expert_body, carry)
        for q in range(2):
            rs = carry[0][q]
            ixs = carry[1][q]
            for j in range(TOP_K):
                ix = ixs[j]
                lo = ix & (L - 1)
                hi = ix >> 4
                b = bias_regs[0].at[lo].get(mode="promise_in_bounds")
                for k in range(1, NUM_EXPERTS // L):
                    gk = bias_regs[k].at[lo].get(mode="promise_in_bounds")
                    b = jnp.where(hi == k, gk, b)
                wj = (rs[j] - b) * ROUTED_SCALING_FACTOR
                idx_v[j, pl.ds(col0 + q * L, L)] = ix
                w_v[j, pl.ds(col0 + q * L, L)] = wj
        return 0

    lax.fori_loop(0, NG // 2, group_body, 0)
    pltpu.sync_copy(idx_v, idx_hbm.at[:, pl.ds(wid * TPW, TPW)])
    pltpu.sync_copy(w_v, w_hbm.at[:, pl.ds(wid * TPW, TPW)])


@jax.jit
def _sc_topk(bt, bias):
    mesh = plsc.VectorSubcoreMesh(core_axis_name="c", subcore_axis_name="s")
    return pl.kernel(
        _sc_body,
        out_type=[
            jax.ShapeDtypeStruct((TOP_K, TOKENS), jnp.int32),
            jax.ShapeDtypeStruct((TOP_K, TOKENS), jnp.float32),
        ],
        mesh=mesh,
        scratch_types=[
            pltpu.VMEM((NUM_EXPERTS, TPW), jnp.float32),
            pltpu.VMEM((NUM_EXPERTS,), jnp.float32),
            pltpu.VMEM((TOP_K, TPW), jnp.int32),
            pltpu.VMEM((TOP_K, TPW), jnp.float32),
        ],
    )(bt, bias)


def kernel(hidden_states, classifier_weight, e_score_correction_bias):
    hs = hidden_states.reshape(-1, HIDDEN).astype(jnp.float32)
    bias_col = e_score_correction_bias.reshape(NUM_EXPERTS, 1)
    bt = _tc_scores(hs, classifier_weight, bias_col)
    idx_rm, w_rm = _sc_topk(bt, e_score_correction_bias)
    return idx_rm.T, w_rm.T
